# baseline (speedup 1.0000x reference)
import jax
import jax.numpy as jnp
from jax.experimental import pallas as pl
from jax.experimental.pallas import tpu as pltpu
from jax.experimental.pallas import tpu_sc as plsc

_ROWS = 32768
_HID = 768
_EXP = 8
_B = 4096

_LANES = 16
_COLS = 512


def _dense_block(w_ref, thr_ref, x_ref, st_ref, wn_ref):
    @pl.when(pl.program_id(0) == 0)
    def _():
        w = w_ref[...]
        wn = w / jnp.maximum(
            jnp.sqrt(jnp.sum(w * w, axis=0, keepdims=True)), 1e-12
        )
        wn_ref[...] = wn.astype(jnp.bfloat16)

    x = x_ref[...]
    ss = jnp.sum(x * x, axis=1, keepdims=True)
    xn = x / jnp.maximum(jnp.sqrt(ss), 1e-12)
    scores = jax.lax.dot_general(
        xn.astype(jnp.bfloat16), wn_ref[...],
        (((1,), (0,)), ((), ())),
        preferred_element_type=jnp.float32,
    )
    st_ref[...] = scores.T


def _dense_scores(hidden_states, sim_matrix, threshold):
    thr2 = threshold.reshape(1, 1)
    return pl.pallas_call(
        _dense_block,
        grid=(_ROWS // _B,),
        in_specs=[
            pl.BlockSpec((_HID, _EXP), lambda i: (0, 0)),
            pl.BlockSpec((1, 1), lambda i: (0, 0)),
            pl.BlockSpec((_B, _HID), lambda i: (i, 0)),
        ],
        out_specs=pl.BlockSpec((_EXP, _B), lambda i: (0, i)),
        out_shape=jax.ShapeDtypeStruct((_EXP, _ROWS), jnp.float32),
        scratch_shapes=[pltpu.VMEM((_HID, _EXP), jnp.bfloat16)],
        compiler_params=pltpu.CompilerParams(
            dimension_semantics=("arbitrary",),
        ),
    )(sim_matrix, thr2, hidden_states)


def _sc_tail(st, thr_arr):
    mesh = plsc.VectorSubcoreMesh(
        core_axis_name="core", subcore_axis_name="subcore"
    )

    @pl.kernel(
        out_type=[
            jax.ShapeDtypeStruct((_EXP, _ROWS), jnp.float32),
            jax.ShapeDtypeStruct((1, _ROWS), jnp.int32),
        ],
        mesh=mesh,
        scratch_types=[],
    )
    def tail_kernel(st_hbm, thr_hbm, rwt_hbm, kt_hbm):
        def body(st_vm, thr_vm, rwt_vm, kt_vm):
            t = thr_vm[0, :]

            @pl.loop(0, _COLS, step=_LANES)
            def _(c):
                sl = pl.ds(c, _LANES)
                e_list = []
                m_list = []
                for j in range(_EXP):
                    s_j = st_vm[j, sl]
                    m_j = jnp.where(s_j > t, 1.0, 0.0)
                    e_j = jnp.exp(s_j - 1.0) * m_j
                    m_list.append(m_j)
                    e_list.append(e_j)
                ssum = e_list[0]
                cnt = m_list[0]
                for j in range(1, _EXP):
                    ssum = ssum + e_list[j]
                    cnt = cnt + m_list[j]
                kt_vm[0, sl] = cnt.astype(jnp.int32)
                inv = 1.0 / ssum
                for j in range(_EXP):
                    rwt_vm[j, sl] = jnp.where(
                        cnt > 0.5, e_list[j] * inv, jnp.float32(0.125)
                    )

        pltpu.emit_pipeline(
            body,
            grid=(_ROWS // _COLS,),
            in_specs=[
                pl.BlockSpec((_EXP, _COLS), lambda i: (0, i)),
                pl.BlockSpec((1, _LANES), lambda i: (0, 0)),
            ],
            out_specs=[
                pl.BlockSpec((_EXP, _COLS), lambda i: (0, i)),
                pl.BlockSpec((1, _COLS), lambda i: (0, i)),
            ],
            core_axis_name=("core", "subcore"),
            dimension_semantics=(pltpu.PARALLEL,),
        )(st_hbm, thr_hbm, rwt_hbm, kt_hbm)

    return tail_kernel(st, thr_arr)


def kernel(hidden_states, sim_matrix, threshold):
    st = _dense_scores(hidden_states, sim_matrix, threshold)
    thr_arr = jnp.broadcast_to(threshold.reshape(1, 1), (1, _LANES))
    rwt, kt = _sc_tail(st, thr_arr)
    return rwt.T, st.T, kt.reshape(_ROWS)


# speedup vs baseline: 1.6322x; 1.6322x over previous
import jax
import jax.numpy as jnp
from jax.experimental import pallas as pl
from jax.experimental.pallas import tpu as pltpu

_ROWS = 32768
_HID = 768
_EXP = 8
_B = 4096


def _gate_block(w_ref, thr_ref, x_ref, rwt_ref, st_ref, kt_ref, wn_ref):
    @pl.when(pl.program_id(0) == 0)
    def _():
        w = w_ref[...]
        wn = w / jnp.maximum(
            jnp.sqrt(jnp.sum(w * w, axis=0, keepdims=True)), 1e-12
        )
        wn_ref[...] = wn.astype(jnp.bfloat16)

    x = x_ref[...]
    ss = jnp.sum(x * x, axis=1, keepdims=True)
    xn = x / jnp.maximum(jnp.sqrt(ss), 1e-12)
    scores = jax.lax.dot_general(
        xn.astype(jnp.bfloat16), wn_ref[...],
        (((1,), (0,)), ((), ())),
        preferred_element_type=jnp.float32,
    )
    st = scores.T
    st_ref[...] = st
    maskf = (st > thr_ref[0, 0]).astype(jnp.float32)
    e = jnp.exp(st - 1.0) * maskf
    ssum = jnp.sum(e, axis=0, keepdims=True)
    cnt = jnp.sum(maskf, axis=0, keepdims=True)
    kt_ref[...] = cnt.astype(jnp.int32)
    rwt_ref[...] = jnp.where(cnt > 0.5, e / ssum, jnp.float32(0.125))


def kernel(hidden_states, sim_matrix, threshold):
    thr2 = threshold.reshape(1, 1)
    rwt, st, kt = pl.pallas_call(
        _gate_block,
        grid=(_ROWS // _B,),
        in_specs=[
            pl.BlockSpec((_HID, _EXP), lambda i: (0, 0)),
            pl.BlockSpec((1, 1), lambda i: (0, 0)),
            pl.BlockSpec((_B, _HID), lambda i: (i, 0)),
        ],
        out_specs=[
            pl.BlockSpec((_EXP, _B), lambda i: (0, i)),
            pl.BlockSpec((_EXP, _B), lambda i: (0, i)),
            pl.BlockSpec((1, _B), lambda i: (0, i)),
        ],
        out_shape=[
            jax.ShapeDtypeStruct((_EXP, _ROWS), jnp.float32),
            jax.ShapeDtypeStruct((_EXP, _ROWS), jnp.float32),
            jax.ShapeDtypeStruct((1, _ROWS), jnp.int32),
        ],
        scratch_shapes=[pltpu.VMEM((_HID, _EXP), jnp.bfloat16)],
        compiler_params=pltpu.CompilerParams(
            dimension_semantics=("arbitrary",),
        ),
    )(sim_matrix, thr2, hidden_states)
    return rwt.T, st.T, kt.reshape(_ROWS)
